# baseline (speedup 1.0000x reference)
import jax
import jax.numpy as jnp
import numpy as np
from jax.experimental import pallas as pl
from jax.experimental.pallas import tpu as pltpu
from jax.sharding import Mesh, PartitionSpec as P

_B = 8192
_BR = 256


def _body(rows_ref, cols_ref, out_ref):
    xr = rows_ref[:, :]
    t1r = jnp.floor(xr)
    t2r = jnp.ceil(xr)
    i1r = t1r.astype(jnp.int32)
    i2r = jnp.minimum(t2r.astype(jnp.int32), _B - 1)

    xc = cols_ref[:, :]
    t1c = jnp.floor(xc)
    tc = jnp.where(jnp.ceil(xc) != t1c, xc - t1c, 0.0)
    w2 = 1.0 - tc

    col = jax.lax.broadcasted_iota(jnp.int32, (8, _B), 1)
    for g in range(_BR // 8):
        s = slice(g * 8, (g + 1) * 8)
        a = col == i1r[s, :]
        b = col == i2r[s, :]
        out_ref[s, :] = jnp.where(a, tc, 0.0) + jnp.where(b, w2, 0.0)


def _masked_write(idx_rows, idx_cols):
    rows = idx_rows.shape[0]
    return pl.pallas_call(
        _body,
        grid=(rows // _BR,),
        in_specs=[
            pl.BlockSpec((_BR, 1), lambda i: (i, 0)),
            pl.BlockSpec((1, _B), lambda i: (0, 0)),
        ],
        out_specs=pl.BlockSpec((_BR, _B), lambda i: (i, 0)),
        out_shape=jax.ShapeDtypeStruct((rows, _B), idx_rows.dtype),
        compiler_params=pltpu.CompilerParams(
            dimension_semantics=("arbitrary",),
        ),
    )(idx_rows, idx_cols)


def kernel(index):
    idx_rows = index.reshape(_B, 1)
    idx_cols = index.reshape(1, _B)
    devs = jax.devices()
    ndev = len(devs)
    if ndev > 1 and _B % (ndev * _BR) == 0:
        mesh = Mesh(np.array(devs), ("x",))
        fn = jax.shard_map(
            _masked_write,
            mesh=mesh,
            in_specs=(P("x", None), P(None, None)),
            out_specs=P("x", None),
            check_vma=False,
        )
        out = fn(idx_rows, idx_cols)
        return jax.lax.with_sharding_constraint(
            out, jax.sharding.NamedSharding(mesh, P("x", None)))
    return _masked_write(idx_rows, idx_cols)


# speedup vs baseline: 5.6553x; 5.6553x over previous
import jax
import jax.numpy as jnp
import numpy as np
from jax.experimental import pallas as pl
from jax.experimental.pallas import tpu as pltpu
from jax.sharding import Mesh, PartitionSpec as P

_B = 8192
_BR = 512


def _body(rows_ref, cols_ref, out_ref):
    xr = rows_ref[:, :]
    t1r = jnp.floor(xr)
    t2r = jnp.ceil(xr)
    i1r = t1r.astype(jnp.int32)
    i2r = jnp.minimum(t2r.astype(jnp.int32), _B - 1)

    xc = cols_ref[:, :]
    t1c = jnp.floor(xc)
    tc = jnp.where(jnp.ceil(xc) != t1c, xc - t1c, 0.0)
    w2 = 1.0 - tc

    col = jax.lax.broadcasted_iota(jnp.int32, (8, _B), 1)
    for g in range(_BR // 8):
        s = slice(g * 8, (g + 1) * 8)
        a = col == i1r[s, :]
        b = col == i2r[s, :]
        out_ref[s, :] = jnp.where(a, tc, 0.0) + jnp.where(b, w2, 0.0)


def _masked_write(idx_rows, idx_cols):
    rows = idx_rows.shape[0]
    return pl.pallas_call(
        _body,
        grid=(rows // _BR,),
        in_specs=[
            pl.BlockSpec((_BR, 1), lambda i: (i, 0)),
            pl.BlockSpec((1, _B), lambda i: (0, 0)),
        ],
        out_specs=pl.BlockSpec((_BR, _B), lambda i: (i, 0)),
        out_shape=jax.ShapeDtypeStruct((rows, _B), idx_rows.dtype),
        compiler_params=pltpu.CompilerParams(
            dimension_semantics=("arbitrary",),
        ),
    )(idx_rows, idx_cols)


def kernel(index):
    idx_rows = index.reshape(_B, 1)
    idx_cols = index.reshape(1, _B)
    return _masked_write(idx_rows, idx_cols)
